# Initial kernel scaffold; baseline (speedup 1.0000x reference)
import functools

import jax
import jax.numpy as jnp
from jax import lax
from jax.experimental import pallas as pl
from jax.experimental.pallas import tpu as pltpu
from jax.experimental.pallas import tpu_sc as plsc

N_NODES = 10000
N_EDGES = 320000
D = 128

NC = 2
NS = 16
NW = NC * NS
EDGES_PER_TILE = N_EDGES // NW
CHUNK = 80
NCHUNK = EDGES_PER_TILE // CHUNK
ROWS_PER_TILE = N_NODES // NS
ZROWS = 125


def _seg_sum_sc():
    mesh = plsc.VectorSubcoreMesh(core_axis_name="c", subcore_axis_name="s")

    @functools.partial(
        pl.kernel,
        mesh=mesh,
        out_type=jax.ShapeDtypeStruct((2 * N_NODES, D), jnp.float32),
        scratch_types=[
            pltpu.VMEM((CHUNK,), jnp.int32),
            pltpu.VMEM((CHUNK,), jnp.int32),
            pltpu.VMEM((CHUNK, D), jnp.float32),
            pltpu.VMEM((ZROWS, D), jnp.float32),
            pltpu.VMEM_SHARED((N_NODES, D), jnp.float32),
            pltpu.SemaphoreType.DMA,
        ],
    )
    def seg_sum(h_hbm, src_hbm, dst_hbm, out_hbm, src_v, dst_v, rows_v,
                zbuf, acc, sem):
        cid = lax.axis_index("c")
        sid = lax.axis_index("s")
        wid = sid * NC + cid

        def _zrow(r, _):
            for j in range(D // 16):
                zbuf[r, pl.ds(j * 16, 16)] = jnp.zeros((16,), jnp.float32)
            return 0

        lax.fori_loop(0, ZROWS, _zrow, 0)
        for t in range(ROWS_PER_TILE // ZROWS):
            pltpu.sync_copy(
                zbuf, acc.at[pl.ds(sid * ROWS_PER_TILE + t * ZROWS, ZROWS)])
        plsc.subcore_barrier()

        ebase = wid * EDGES_PER_TILE

        def _chunk(j, _):
            off = ebase + j * CHUNK
            pltpu.sync_copy(src_hbm.at[pl.ds(off, CHUNK)], src_v)
            pltpu.sync_copy(dst_hbm.at[pl.ds(off, CHUNK)], dst_v)
            pltpu.async_copy(h_hbm.at[src_v], rows_v, sem).wait()
            pltpu.sync_copy(rows_v, acc.at[dst_v], add=True)
            return 0

        lax.fori_loop(0, NCHUNK, _chunk, 0)
        plsc.subcore_barrier()

        pltpu.sync_copy(
            acc.at[pl.ds(sid * ROWS_PER_TILE, ROWS_PER_TILE)],
            out_hbm.at[pl.ds(cid * N_NODES + sid * ROWS_PER_TILE,
                             ROWS_PER_TILE)])

    return seg_sum


_SEG_SUM = _seg_sum_sc()


def _mm_body(x_ref, w_ref, o_ref):
    o_ref[...] = jnp.dot(x_ref[...], w_ref[...],
                         preferred_element_type=jnp.float32)


def _matmul(x, w):
    return pl.pallas_call(
        _mm_body,
        out_shape=jax.ShapeDtypeStruct((x.shape[0], w.shape[1]), jnp.float32),
    )(x, w)


def _bn_relu_mm_body(p_ref, b_ref, g_ref, be_ref, w_ref, o_ref):
    s = p_ref[0:N_NODES, :] + p_ref[N_NODES:2 * N_NODES, :] + b_ref[...]
    mean = jnp.mean(s, axis=0, keepdims=True)
    d0 = s - mean
    var = jnp.mean(d0 * d0, axis=0, keepdims=True)
    y = d0 * lax.rsqrt(var + 1e-5) * g_ref[...] + be_ref[...]
    y = jnp.maximum(y, 0.0)
    o_ref[...] = jnp.dot(y, w_ref[...], preferred_element_type=jnp.float32)


def _bn_relu_mm(p, b, g, be, w):
    return pl.pallas_call(
        _bn_relu_mm_body,
        out_shape=jax.ShapeDtypeStruct((N_NODES, D), jnp.float32),
    )(p, b.reshape(1, D), g.reshape(1, D), be.reshape(1, D), w)


def _final_body(p_ref, b_ref, o_ref):
    o_ref[...] = p_ref[0:N_NODES, :] + p_ref[N_NODES:2 * N_NODES, :] + b_ref[...]


def _final(p, b):
    return pl.pallas_call(
        _final_body,
        out_shape=jax.ShapeDtypeStruct((N_NODES, D), jnp.float32),
    )(p, b.reshape(1, D))


def kernel(x, edge_index, W1, b1, W2, b2, W3, b3, g1, be1, g2, be2):
    src = edge_index[0]
    dst = edge_index[1]
    h = _matmul(x, W1)
    p = _SEG_SUM(h, src, dst)
    h = _bn_relu_mm(p, b1, g1, be1, W2)
    p = _SEG_SUM(h, src, dst)
    h = _bn_relu_mm(p, b2, g2, be2, W3)
    p = _SEG_SUM(h, src, dst)
    return _final(p, b3)


# speedup vs baseline: 4.6510x; 4.6510x over previous
import functools

import jax
import jax.numpy as jnp
from jax import lax
from jax.experimental import pallas as pl
from jax.experimental.pallas import tpu as pltpu
from jax.experimental.pallas import tpu_sc as plsc

N_NODES = 10000
N_EDGES = 320000
D = 128

NC = 2
NS = 16
NW = NC * NS
EDGES_PER_TILE = N_EDGES // NW
CHUNK = 80
NCHUNK = EDGES_PER_TILE // CHUNK
ACC_ROWS = 10240
ROWS_PER_TILE = ACC_ROWS // NS
ZROWS = 128


def _seg_sum_sc():
    mesh = plsc.VectorSubcoreMesh(core_axis_name="c", subcore_axis_name="s")

    @functools.partial(
        pl.kernel,
        mesh=mesh,
        out_type=jax.ShapeDtypeStruct((2 * ACC_ROWS, D), jnp.float32),
        scratch_types=[
            pltpu.VMEM((CHUNK,), jnp.int32),
            pltpu.VMEM((CHUNK,), jnp.int32),
            pltpu.VMEM((CHUNK, D), jnp.float32),
            pltpu.VMEM((ZROWS, D), jnp.float32),
            pltpu.VMEM_SHARED((ACC_ROWS, D), jnp.float32),
            pltpu.SemaphoreType.DMA,
        ],
    )
    def seg_sum(h_hbm, src_hbm, dst_hbm, out_hbm, src_v, dst_v, rows_v,
                zbuf, acc, sem):
        cid = lax.axis_index("c")
        sid = lax.axis_index("s")
        wid = sid * NC + cid

        def _zrow(r, _):
            for j in range(D // 16):
                zbuf[r, pl.ds(j * 16, 16)] = jnp.zeros((16,), jnp.float32)
            return 0

        lax.fori_loop(0, ZROWS, _zrow, 0)
        for t in range(ROWS_PER_TILE // ZROWS):
            pltpu.sync_copy(
                zbuf, acc.at[pl.ds(sid * ROWS_PER_TILE + t * ZROWS, ZROWS)])
        plsc.subcore_barrier()

        ebase = wid * EDGES_PER_TILE

        def _chunk(j, _):
            off = ebase + j * CHUNK
            pltpu.sync_copy(src_hbm.at[pl.ds(off, CHUNK)], src_v)
            pltpu.sync_copy(dst_hbm.at[pl.ds(off, CHUNK)], dst_v)
            pltpu.async_copy(h_hbm.at[src_v], rows_v, sem).wait()
            pltpu.sync_copy(rows_v, acc.at[dst_v], add=True)
            return 0

        lax.fori_loop(0, NCHUNK, _chunk, 0)
        plsc.subcore_barrier()

        pltpu.sync_copy(
            acc.at[pl.ds(sid * ROWS_PER_TILE, ROWS_PER_TILE)],
            out_hbm.at[pl.ds(cid * ACC_ROWS + sid * ROWS_PER_TILE,
                             ROWS_PER_TILE)])

    return seg_sum


_SEG_SUM = _seg_sum_sc()


def _mm_body(x_ref, w_ref, o_ref):
    o_ref[...] = jnp.dot(x_ref[...], w_ref[...],
                         preferred_element_type=jnp.float32)


def _matmul(x, w):
    return pl.pallas_call(
        _mm_body,
        out_shape=jax.ShapeDtypeStruct((x.shape[0], w.shape[1]), jnp.float32),
    )(x, w)


def _bn_relu_mm_body(p_ref, b_ref, g_ref, be_ref, w_ref, o_ref):
    s = p_ref[0:N_NODES, :] + p_ref[ACC_ROWS:ACC_ROWS + N_NODES, :] + b_ref[...]
    mean = jnp.mean(s, axis=0, keepdims=True)
    d0 = s - mean
    var = jnp.mean(d0 * d0, axis=0, keepdims=True)
    y = d0 * lax.rsqrt(var + 1e-5) * g_ref[...] + be_ref[...]
    y = jnp.maximum(y, 0.0)
    o_ref[...] = jnp.dot(y, w_ref[...], preferred_element_type=jnp.float32)


def _bn_relu_mm(p, b, g, be, w):
    return pl.pallas_call(
        _bn_relu_mm_body,
        out_shape=jax.ShapeDtypeStruct((N_NODES, D), jnp.float32),
    )(p, b.reshape(1, D), g.reshape(1, D), be.reshape(1, D), w)


def _final_body(p_ref, b_ref, o_ref):
    o_ref[...] = p_ref[0:N_NODES, :] + p_ref[ACC_ROWS:ACC_ROWS + N_NODES, :] + b_ref[...]


def _final(p, b):
    return pl.pallas_call(
        _final_body,
        out_shape=jax.ShapeDtypeStruct((N_NODES, D), jnp.float32),
    )(p, b.reshape(1, D))


def kernel(x, edge_index, W1, b1, W2, b2, W3, b3, g1, be1, g2, be2):
    src = edge_index[0]
    dst = edge_index[1]
    h = _matmul(x, W1)
    p = _SEG_SUM(h, src, dst)
    h = _bn_relu_mm(p, b1, g1, be1, W2)
    p = _SEG_SUM(h, src, dst)
    h = _bn_relu_mm(p, b2, g2, be2, W3)
    p = _SEG_SUM(h, src, dst)
    return _final(p, b3)
